# baseline (speedup 1.0000x reference)
import jax
import jax.numpy as jnp
from jax import lax
from jax.experimental import pallas as pl
from jax.experimental.pallas import tpu as pltpu, tpu_sc as plsc

H = 64
IN = 2
GX, GY = 4, 4
NS = 4.0
OBS = 15
N = 1024
G = GX * GY

_DN_BT = (((1,), (1,)), ((), ()))

NW = 32
NPW = N // NW


def _stage_a1(An_ref, AW_ref, ho_ref, idx_ref):
    f32 = jnp.float32
    others = An_ref[0:N, 0:IN]
    maskf = An_ref[0:N, IN:IN + 1]
    WihT = AW_ref[0:IN, :]
    b_row = AW_ref[IN:IN + 1, :]
    tpos = AW_ref[IN + 1:IN + 2, 0:2]

    gates_o = jnp.dot(others, WihT, preferred_element_type=f32) + b_row
    co = (jax.nn.sigmoid(gates_o[:, 0:H])
          * jnp.tanh(gates_o[:, 2 * H:3 * H]))
    ho_ref[...] = (jax.nn.sigmoid(gates_o[:, 3 * H:4 * H])
                   * jnp.tanh(co))

    cell_w = NS / GX
    cell_h = NS / GY
    px = tpos[0:1, 0:1]
    py = tpos[0:1, 1:2]
    rx = others[:, 0:1] - px
    ry = others[:, 1:2] - py
    within = (jnp.abs(rx) <= NS / 2) & (jnp.abs(ry) <= NS / 2)
    cx = (rx / cell_w).astype(jnp.int32) + GX // 2
    cy = (ry / cell_h).astype(jnp.int32) + GY // 2
    inb = (cx >= 0) & (cx < GX) & (cy >= 0) & (cy < GY)
    m = within & inb & (maskf != 0.0)
    idx_ref[...] = jnp.where(m, cy * GX + cx, G)


def _stage_a2(B64_ref, C_ref, h_ref):
    f32 = jnp.float32
    Whh = B64_ref[0:4 * H, :]
    Wih = B64_ref[4 * H + 8:8 * H + 8, 0:IN]
    targetT = B64_ref[8 * H + 8:8 * H + 10, 0:OBS]
    b = C_ref[0:4 * H, :]
    xg = jnp.dot(Wih, targetT, preferred_element_type=f32) + b
    h = jnp.zeros((H, 1), f32)
    c = jnp.zeros((H, 1), f32)
    for t in range(OBS):
        gates = xg[:, t:t + 1] + jnp.dot(Whh, h, preferred_element_type=f32)
        gi = jax.nn.sigmoid(gates[0:H, :])
        gf = jax.nn.sigmoid(gates[H:2 * H, :])
        gg = jnp.tanh(gates[2 * H:3 * H, :])
        go = jax.nn.sigmoid(gates[3 * H:4 * H, :])
        c = gf * c + gi * gg
        h = go * jnp.tanh(c)
    h_ref[...] = h


def _stage_b(ho_hbm, idx_hbm, part_hbm, rows_v, idx_v, grid_v):
    cid = lax.axis_index("c")
    sid = lax.axis_index("s")
    wid = sid * 2 + cid
    base = wid * NPW
    pltpu.sync_copy(ho_hbm.at[pl.ds(base * H, NPW * H)], rows_v)
    pltpu.sync_copy(idx_hbm.at[pl.ds(base, NPW + 16)], idx_v)
    zeros = jnp.zeros((16,), jnp.float32)
    for i in range((G + 1) * H // 16):
        grid_v[pl.ds(i * 16, 16)] = zeros

    def body(n, carry):
        s = idx_v[pl.ds(n, 16)][0]
        row0 = s * H
        for l in range(H // 16):
            dst = pl.ds(row0 + l * 16, 16)
            grid_v[dst] = grid_v[dst] + rows_v[pl.ds(n * H + l * 16, 16)]
        return carry

    lax.fori_loop(0, NPW, body, jnp.int32(0))
    pltpu.sync_copy(grid_v.at[pl.ds(0, G * H)], part_hbm.at[wid])


def _scatter_sc(ho, idx_pad):
    mesh = plsc.VectorSubcoreMesh(core_axis_name="c", subcore_axis_name="s",
                                  num_cores=2, num_subcores=16)
    f = pl.kernel(
        _stage_b,
        out_type=jax.ShapeDtypeStruct((NW, G * H), jnp.float32),
        mesh=mesh,
        scratch_types=[
            pltpu.VMEM((NPW * H,), jnp.float32),
            pltpu.VMEM((NPW + 16,), jnp.int32),
            pltpu.VMEM(((G + 1) * H,), jnp.float32),
        ],
    )
    return f(ho.reshape(N * H), idx_pad)


def _stage_c(Cw_ref, D64_ref, out_ref):
    f32 = jnp.float32
    W1 = Cw_ref[0:H, :]
    parts = Cw_ref[H:H + NW, :]
    b1r = Cw_ref[H + NW:H + NW + 1, 0:H]
    b2r = Cw_ref[H + NW + 1:H + NW + 2, 0:H]
    hr = Cw_ref[H + NW + 2:H + NW + 3, 0:H]
    bcr = Cw_ref[H + NW + 3:H + NW + 4, 0:2]
    W2 = D64_ref[0:H, :]
    Wc = D64_ref[H:H + 2, :]

    st = jnp.sum(parts, axis=0, keepdims=True)
    acc = jax.lax.dot_general(st, W1, _DN_BT,
                              preferred_element_type=f32) + b1r
    sc = (jax.lax.dot_general(jnp.maximum(acc, 0.0), W2, _DN_BT,
                              preferred_element_type=f32) + b2r)
    out_ref[...] = (jax.lax.dot_general(hr + sc, Wc, _DN_BT,
                                        preferred_element_type=f32) + bcr)


def kernel(observed_trajectory_target, observed_trajectory_others, neighbor_mask,
           W_ih, W_hh, b_ih, b_hh, W1, b1, W2, b2, Wc, bc):
    others = observed_trajectory_others[OBS - 1]
    maskf = neighbor_mask[OBS - 1].astype(jnp.float32)[:, None]
    b_comb = b_ih + b_hh
    An = jnp.concatenate([others, maskf], axis=1)
    AW = jnp.concatenate([
        W_ih.T, b_comb[None, :],
        jnp.pad(observed_trajectory_target[OBS - 1][None, :],
                ((0, 0), (0, 4 * H - IN))),
    ], axis=0)
    ho, idx = pl.pallas_call(
        _stage_a1,
        out_shape=(
            jax.ShapeDtypeStruct((N, H), jnp.float32),
            jax.ShapeDtypeStruct((N, 1), jnp.int32),
        ),
    )(An, AW)

    idx_pad = jnp.concatenate([idx.reshape(N), jnp.zeros((16,), jnp.int32)])
    parts = _scatter_sc(ho, idx_pad)

    B64 = jnp.concatenate([
        W_hh,
        jnp.zeros((8, H), jnp.float32),
        jnp.pad(W_ih, ((0, 0), (0, H - IN))),
        jnp.pad(observed_trajectory_target.T, ((0, 6), (0, H - OBS))),
    ], axis=0)
    h = pl.pallas_call(
        _stage_a2,
        out_shape=jax.ShapeDtypeStruct((H, 1), jnp.float32),
    )(B64, b_comb[:, None])

    pad_row = lambda v: jnp.pad(v.reshape(1, -1),
                                ((0, 0), (0, G * H - v.size)))
    Cw = jnp.concatenate([
        W1, parts, pad_row(b1), pad_row(b2), pad_row(h), pad_row(bc),
    ], axis=0)
    D64 = jnp.concatenate([W2, Wc], axis=0)
    out = pl.pallas_call(
        _stage_c,
        out_shape=jax.ShapeDtypeStruct((1, 2), jnp.float32),
    )(Cw, D64)
    return out


# speedup vs baseline: 1.0929x; 1.0929x over previous
import jax
import jax.numpy as jnp
from jax import lax
from jax.experimental import pallas as pl
from jax.experimental.pallas import tpu as pltpu, tpu_sc as plsc

H = 64
IN = 2
GX, GY = 4, 4
NS = 4.0
OBS = 15
N = 1024
G = GX * GY

_DN_BT = (((1,), (1,)), ((), ()))

NW = 32
NPW = N // NW


def _stage_a(An_ref, AW_ref, B64_ref, ho_ref, idx_ref, h_ref):
    f32 = jnp.float32
    others = An_ref[0:N, 0:IN]
    maskf = An_ref[0:N, IN:IN + 1]
    b = An_ref[N:N + 4 * H, 0:1]
    WihT = AW_ref[0:IN, :]
    b_row = AW_ref[IN:IN + 1, :]
    Whh = B64_ref[0:4 * H, :]
    Wih = B64_ref[4 * H + 8:8 * H + 8, 0:IN]
    targetT = B64_ref[8 * H + 8:8 * H + 10, 0:OBS]

    xg = jnp.dot(Wih, targetT, preferred_element_type=f32) + b
    h = jnp.zeros((H, 1), f32)
    c = jnp.zeros((H, 1), f32)
    for t in range(OBS):
        gates = xg[:, t:t + 1] + jnp.dot(Whh, h, preferred_element_type=f32)
        gi = jax.nn.sigmoid(gates[0:H, :])
        gf = jax.nn.sigmoid(gates[H:2 * H, :])
        gg = jnp.tanh(gates[2 * H:3 * H, :])
        go = jax.nn.sigmoid(gates[3 * H:4 * H, :])
        c = gf * c + gi * gg
        h = go * jnp.tanh(c)
    h_ref[...] = h

    gates_o = jnp.dot(others, WihT, preferred_element_type=f32) + b_row
    co = (jax.nn.sigmoid(gates_o[:, 0:H])
          * jnp.tanh(gates_o[:, 2 * H:3 * H]))
    ho_ref[...] = (jax.nn.sigmoid(gates_o[:, 3 * H:4 * H])
                   * jnp.tanh(co))

    cell_w = NS / GX
    cell_h = NS / GY
    px = targetT[0:1, OBS - 1:OBS]
    py = targetT[1:2, OBS - 1:OBS]
    rx = others[:, 0:1] - px
    ry = others[:, 1:2] - py
    within = (jnp.abs(rx) <= NS / 2) & (jnp.abs(ry) <= NS / 2)
    cx = (rx / cell_w).astype(jnp.int32) + GX // 2
    cy = (ry / cell_h).astype(jnp.int32) + GY // 2
    inb = (cx >= 0) & (cx < GX) & (cy >= 0) & (cy < GY)
    m = within & inb & (maskf != 0.0)
    idx_ref[...] = jnp.where(m, cy * GX + cx, G)


def _stage_b(ho_hbm, idx_hbm, part_hbm, rows_v, idx_v, grid_v):
    cid = lax.axis_index("c")
    sid = lax.axis_index("s")
    wid = sid * 2 + cid
    base = wid * NPW
    pltpu.sync_copy(ho_hbm.at[pl.ds(base * H, NPW * H)], rows_v)
    pltpu.sync_copy(idx_hbm.at[pl.ds(base, NPW + 16)], idx_v)
    zeros = jnp.zeros((16,), jnp.float32)
    for i in range((G + 1) * H // 16):
        grid_v[pl.ds(i * 16, 16)] = zeros

    def body(n, carry):
        s = idx_v[pl.ds(n, 16)][0]
        row0 = s * H
        for l in range(H // 16):
            dst = pl.ds(row0 + l * 16, 16)
            grid_v[dst] = grid_v[dst] + rows_v[pl.ds(n * H + l * 16, 16)]
        return carry

    lax.fori_loop(0, NPW, body, jnp.int32(0))
    pltpu.sync_copy(grid_v.at[pl.ds(0, G * H)], part_hbm.at[wid])


def _scatter_sc(ho, idx_pad):
    mesh = plsc.VectorSubcoreMesh(core_axis_name="c", subcore_axis_name="s",
                                  num_cores=2, num_subcores=16)
    f = pl.kernel(
        _stage_b,
        out_type=jax.ShapeDtypeStruct((NW, G * H), jnp.float32),
        mesh=mesh,
        scratch_types=[
            pltpu.VMEM((NPW * H,), jnp.float32),
            pltpu.VMEM((NPW + 16,), jnp.int32),
            pltpu.VMEM(((G + 1) * H,), jnp.float32),
        ],
    )
    return f(ho.reshape(N * H), idx_pad)


def _stage_c(Cw_ref, D64_ref, out_ref):
    f32 = jnp.float32
    W1 = Cw_ref[0:H, :]
    parts = Cw_ref[H:H + NW, :]
    b1r = Cw_ref[H + NW:H + NW + 1, 0:H]
    b2r = Cw_ref[H + NW + 1:H + NW + 2, 0:H]
    hr = Cw_ref[H + NW + 2:H + NW + 3, 0:H]
    bcr = Cw_ref[H + NW + 3:H + NW + 4, 0:2]
    W2 = D64_ref[0:H, :]
    Wc = D64_ref[H:H + 2, :]

    st = jnp.sum(parts, axis=0, keepdims=True)
    acc = jax.lax.dot_general(st, W1, _DN_BT,
                              preferred_element_type=f32) + b1r
    sc = (jax.lax.dot_general(jnp.maximum(acc, 0.0), W2, _DN_BT,
                              preferred_element_type=f32) + b2r)
    out_ref[...] = (jax.lax.dot_general(hr + sc, Wc, _DN_BT,
                                        preferred_element_type=f32) + bcr)


def kernel(observed_trajectory_target, observed_trajectory_others, neighbor_mask,
           W_ih, W_hh, b_ih, b_hh, W1, b1, W2, b2, Wc, bc):
    others = observed_trajectory_others[OBS - 1]
    maskf = neighbor_mask[OBS - 1].astype(jnp.float32)[:, None]
    b_comb = b_ih + b_hh
    An = jnp.concatenate([
        jnp.concatenate([others, maskf], axis=1),
        jnp.pad(b_comb[:, None], ((0, 0), (0, 2))),
    ], axis=0)
    AW = jnp.concatenate([W_ih.T, b_comb[None, :]], axis=0)
    B64 = jnp.concatenate([
        W_hh,
        jnp.zeros((8, H), jnp.float32),
        jnp.pad(W_ih, ((0, 0), (0, H - IN))),
        jnp.pad(observed_trajectory_target.T, ((0, 6), (0, H - OBS))),
    ], axis=0)
    ho, idx, h = pl.pallas_call(
        _stage_a,
        out_shape=(
            jax.ShapeDtypeStruct((N, H), jnp.float32),
            jax.ShapeDtypeStruct((N, 1), jnp.int32),
            jax.ShapeDtypeStruct((H, 1), jnp.float32),
        ),
    )(An, AW, B64)

    idx_pad = jnp.concatenate([idx.reshape(N), jnp.zeros((16,), jnp.int32)])
    parts = _scatter_sc(ho, idx_pad)

    pad_row = lambda v: jnp.pad(v.reshape(1, -1),
                                ((0, 0), (0, G * H - v.size)))
    Cw = jnp.concatenate([
        W1, parts, pad_row(b1), pad_row(b2), pad_row(h), pad_row(bc),
    ], axis=0)
    D64 = jnp.concatenate([W2, Wc], axis=0)
    out = pl.pallas_call(
        _stage_c,
        out_shape=jax.ShapeDtypeStruct((1, 2), jnp.float32),
    )(Cw, D64)
    return out
